# Initial kernel scaffold; baseline (speedup 1.0000x reference)
import jax
import jax.numpy as jnp
from jax.experimental import pallas as pl


def kernel(inclination, sky_rot, line_broadening, velocity_shift, x0, y0, distance_pc):
    raise NotImplementedError("write your pallas kernel here")


# speedup vs baseline: 57.5496x; 57.5496x over previous
import math

import jax
import jax.numpy as jnp
from jax import lax
from jax.experimental import pallas as pl
from jax.experimental.pallas import tpu as pltpu

N_PIX_LO = 128
OVERSAMP_XY = 4
N_PIX_HI = N_PIX_LO * OVERSAMP_XY
NV_LO = 64
OVERSAMP_V = 4
NV_HI = NV_LO * OVERSAMP_V
K_VEL = 8
PIXSCALE_LO = 0.05
PIXSCALE_HI = PIXSCALE_LO / OVERSAMP_XY
DV_LO = 10.0
DV_HI = DV_LO / OVERSAMP_V
VEL0_LO = -0.5 * (NV_LO - 1) * DV_LO
VEL0_HI = VEL0_LO - 0.5 * (DV_LO - DV_HI)
FOV_HALF_HI = 0.5 * (N_PIX_HI - 1) * PIXSCALE_HI
THETA_E = 1.0
R_D = 500.0
V_MAX = 200.0
R_T = 200.0

def _atan_pos(z):
    t_hi = 2.414213562373095
    t_lo = 0.4142135623730950
    hi = z > t_hi
    mid = z > t_lo
    x = jnp.where(hi, -1.0 / z, jnp.where(mid, (z - 1.0) / (z + 1.0), z))
    w = jnp.where(hi, math.pi / 2.0, jnp.where(mid, math.pi / 4.0, 0.0))
    s = x * x
    p = (((8.05374449538e-2 * s - 1.38776856032e-1) * s + 1.99777106478e-1) * s
         - 3.33329491539e-1) * s * x + x
    return w + p


_LROWS = 8
_ROWS = 4


def _body(params_ref, px_ref, out_ref):
    i = pl.program_id(0)
    f32 = jnp.float32

    cos_i = params_ref[0]
    sin_i = params_ref[1]
    cos_pa = params_ref[2]
    sin_pa = params_ref[3]
    arcsec_per_pc = params_ref[4]
    x0 = params_ref[5]
    y0 = params_ref[6]
    vshift = params_ref[7]

    jj = lax.broadcasted_iota(jnp.int32, (NV_LO, 1, 1), 0)

    for lr in range(_LROWS):
        col = lax.broadcasted_iota(jnp.int32, (_ROWS, N_PIX_HI), 1).astype(f32)
        row = (lax.broadcasted_iota(jnp.int32, (_ROWS, N_PIX_HI), 0)
               + (i * _LROWS + lr) * _ROWS).astype(f32)
        thx = -FOV_HALF_HI + PIXSCALE_HI * col
        thy = -FOV_HALF_HI + PIXSCALE_HI * row

        r = jnp.sqrt(thx * thx + thy * thy) + 1e-12
        bx = thx - THETA_E * thx / r
        by = thy - THETA_E * thy / r
        X = (bx - x0) / arcsec_per_pc
        Y = (by - y0) / arcsec_per_pc
        x_gal = cos_pa * X + sin_pa * Y
        y_gal = (-sin_pa * X + cos_pa * Y) / (cos_i + 1e-12)
        R = jnp.hypot(x_gal, y_gal)
        I_map = jnp.exp(-R / R_D)
        v_circ = V_MAX * (2.0 / math.pi) * _atan_pos(R * (1.0 / R_T))
        cos_theta = x_gal / (R + 1e-12)
        v_los = v_circ * sin_i * cos_theta + vshift

        val = I_map * (1.0 / (K_VEL * OVERSAMP_V * OVERSAMP_XY * OVERSAMP_XY))

        acc = jnp.zeros((NV_LO, _ROWS, N_PIX_HI), f32)
        for k in range(K_VEL):
            dvk = params_ref[8 + k]
            c = ((v_los + dvk) - VEL0_HI) / DV_HI
            iv0 = jnp.clip(jnp.floor(c), 0.0, float(NV_HI - 1))
            fv = jnp.clip(c - iv0, 0.0, 1.0)
            iv0i = iv0.astype(jnp.int32)
            j0 = iv0i >> 2
            j1 = jnp.minimum(iv0i + 1, NV_HI - 1) >> 2
            w0 = val * (1.0 - fv)
            w1 = val * fv
            acc = acc + jnp.where(jj == j0[None], w0[None], 0.0)
            acc = acc + jnp.where(jj == j1[None], w1[None], 0.0)

        acc2 = jnp.sum(acc, axis=1)
        pooled = jnp.dot(acc2, px_ref[...], preferred_element_type=f32)
        out_ref[:, lr, :] = pooled


def kernel(inclination, sky_rot, line_broadening, velocity_shift, x0, y0, distance_pc):
    f32 = jnp.float32
    cos_i = jnp.cos(inclination)
    sin_i = jnp.sin(inclination)
    pa = sky_rot + math.pi / 2.0
    cos_pa = jnp.cos(pa)
    sin_pa = jnp.sin(pa)
    arcsec_per_pc = 206265.0 / distance_pc

    sigma = jnp.abs(line_broadening) + 1e-12
    p_mid = (jnp.arange(K_VEL, dtype=f32) + 0.5) / K_VEL
    unit = math.sqrt(2.0) * jax.scipy.special.erfinv(2.0 * p_mid - 1.0)
    dv_off = sigma * unit

    params = jnp.concatenate([
        jnp.stack([cos_i, sin_i, cos_pa, sin_pa, arcsec_per_pc,
                   x0, y0, velocity_shift]).astype(f32),
        dv_off.astype(f32),
    ])

    px = (lax.broadcasted_iota(jnp.int32, (N_PIX_HI, N_PIX_LO), 0) // OVERSAMP_XY
          == lax.broadcasted_iota(jnp.int32, (N_PIX_HI, N_PIX_LO), 1)).astype(f32)

    out = pl.pallas_call(
        _body,
        grid=(N_PIX_LO // _LROWS,),
        in_specs=[
            pl.BlockSpec(memory_space=pltpu.SMEM),
            pl.BlockSpec((N_PIX_HI, N_PIX_LO), lambda i: (0, 0)),
        ],
        out_specs=pl.BlockSpec((NV_LO, _LROWS, N_PIX_LO), lambda i: (0, i, 0)),
        out_shape=jax.ShapeDtypeStruct((NV_LO, N_PIX_LO, N_PIX_LO), f32),
    )(params, px)
    return out


# speedup vs baseline: 153.6511x; 2.6699x over previous
import functools
import math

import jax
import jax.numpy as jnp
from jax import lax
from jax.experimental import pallas as pl
from jax.experimental.pallas import tpu as pltpu
from jax.experimental.pallas import tpu_sc as plsc

N_PIX_LO = 128
OVERSAMP_XY = 4
N_PIX_HI = N_PIX_LO * OVERSAMP_XY
NV_LO = 64
OVERSAMP_V = 4
NV_HI = NV_LO * OVERSAMP_V
K_VEL = 8
PIXSCALE_LO = 0.05
PIXSCALE_HI = PIXSCALE_LO / OVERSAMP_XY
DV_LO = 10.0
DV_HI = DV_LO / OVERSAMP_V
VEL0_LO = -0.5 * (NV_LO - 1) * DV_LO
VEL0_HI = VEL0_LO - 0.5 * (DV_LO - DV_HI)
FOV_HALF_HI = 0.5 * (N_PIX_HI - 1) * PIXSCALE_HI
THETA_E = 1.0
R_D = 500.0
V_MAX = 200.0
R_T = 200.0

_NC = 2
_NS = 16
_NW = _NC * _NS
_ROWS_W = N_PIX_HI // _NW
_LROWS_W = _ROWS_W // OVERSAMP_XY
_GROUPS = _ROWS_W * (N_PIX_HI // 16)
_HSIZE = NV_LO * _LROWS_W * N_PIX_LO


def _atan_pos(z):
    t_hi = 2.414213562373095
    t_lo = 0.4142135623730950
    hi = z > t_hi
    mid = z > t_lo
    x = jnp.where(hi, -1.0 / z, jnp.where(mid, (z - 1.0) / (z + 1.0), z))
    w = jnp.where(hi, math.pi / 2.0, jnp.where(mid, math.pi / 4.0, 0.0))
    s = x * x
    p = (((8.05374449538e-2 * s - 1.38776856032e-1) * s + 1.99777106478e-1) * s
         - 3.33329491539e-1) * s * x + x
    return w + p


_FROWS = 64


def _fields_body(params_ref, v_ref, a_ref):
    i = pl.program_id(0)
    f32 = jnp.float32

    cos_i = params_ref[0]
    sin_i = params_ref[1]
    cos_pa = params_ref[2]
    sin_pa = params_ref[3]
    arcsec_per_pc = params_ref[4]
    x0 = params_ref[5]
    y0 = params_ref[6]
    vshift = params_ref[7]

    col = lax.broadcasted_iota(jnp.int32, (_FROWS, N_PIX_HI), 1).astype(f32)
    row = (lax.broadcasted_iota(jnp.int32, (_FROWS, N_PIX_HI), 0)
           + i * _FROWS).astype(f32)
    thx = -FOV_HALF_HI + PIXSCALE_HI * col
    thy = -FOV_HALF_HI + PIXSCALE_HI * row

    r = jnp.sqrt(thx * thx + thy * thy) + 1e-12
    bx = thx - THETA_E * thx / r
    by = thy - THETA_E * thy / r
    X = (bx - x0) / arcsec_per_pc
    Y = (by - y0) / arcsec_per_pc
    x_gal = cos_pa * X + sin_pa * Y
    y_gal = (-sin_pa * X + cos_pa * Y) / (cos_i + 1e-12)
    R = jnp.hypot(x_gal, y_gal)
    I_map = jnp.exp(-R / R_D)
    v_circ = V_MAX * (2.0 / math.pi) * _atan_pos(R * (1.0 / R_T))
    cos_theta = x_gal / (R + 1e-12)
    v_ref[...] = v_circ * sin_i * cos_theta + vshift
    a_ref[...] = I_map * (1.0 / (K_VEL * OVERSAMP_V * OVERSAMP_XY * OVERSAMP_XY))


def _fields(params):
    f32 = jnp.float32
    return pl.pallas_call(
        _fields_body,
        grid=(N_PIX_HI // _FROWS,),
        in_specs=[pl.BlockSpec(memory_space=pltpu.SMEM)],
        out_specs=[
            pl.BlockSpec((_FROWS, N_PIX_HI), lambda i: (i, 0)),
            pl.BlockSpec((_FROWS, N_PIX_HI), lambda i: (i, 0)),
        ],
        out_shape=[
            jax.ShapeDtypeStruct((N_PIX_HI, N_PIX_HI), f32),
            jax.ShapeDtypeStruct((N_PIX_HI, N_PIX_HI), f32),
        ],
    )(params)


def _sc_bin_body(v_hbm, a_hbm, dv_hbm, zer_hbm, out_hbm, v_v, a_v, dv_v, hist):
    f32 = jnp.float32
    i32 = jnp.int32
    wid = lax.axis_index("s") * _NC + lax.axis_index("c")
    row0 = wid * _ROWS_W

    pltpu.sync_copy(v_hbm.at[pl.ds(row0, _ROWS_W), :], v_v)
    pltpu.sync_copy(a_hbm.at[pl.ds(row0, _ROWS_W), :], a_v)
    pltpu.sync_copy(dv_hbm, dv_v)
    pltpu.sync_copy(zer_hbm, hist)

    lane = lax.broadcasted_iota(i32, (16,), 0)

    def group(g, carry):
        hi_row = g >> 5
        x0 = pl.multiple_of((g & 31) << 4, 16)
        base = (((hi_row >> 2) << 7) + ((x0 + lane) >> 2)).astype(i32)
        v = v_v[hi_row, pl.ds(x0, 16)]
        a = a_v[hi_row, pl.ds(x0, 16)]
        for k in range(K_VEL):
            dvk = dv_v[k, :]
            c = ((v + dvk) - VEL0_HI) / DV_HI
            iv0 = jnp.clip(c.astype(i32), 0, NV_HI - 1)
            fv = jnp.clip(c - iv0.astype(f32), 0.0, 1.0)
            idx0 = ((iv0 >> 2) << 9) + base
            idx1 = ((jnp.minimum(iv0 + 1, NV_HI - 1) >> 2) << 9) + base
            plsc.addupdate_scatter(hist, [idx0], a * (1.0 - fv))
            plsc.addupdate_scatter(hist, [idx1], a * fv)
        return carry

    lax.fori_loop(0, _GROUPS, group, 0)

    pltpu.sync_copy(hist, out_hbm.at[pl.ds(wid * _HSIZE, _HSIZE)])


@functools.partial(
    pl.kernel,
    mesh=plsc.VectorSubcoreMesh(core_axis_name="c", subcore_axis_name="s"),
    compiler_params=pltpu.CompilerParams(needs_layout_passes=False),
    out_type=jax.ShapeDtypeStruct((_NW * _HSIZE,), jnp.float32),
    scratch_types=[
        pltpu.VMEM((_ROWS_W, N_PIX_HI), jnp.float32),
        pltpu.VMEM((_ROWS_W, N_PIX_HI), jnp.float32),
        pltpu.VMEM((K_VEL, 16), jnp.float32),
        pltpu.VMEM((_HSIZE,), jnp.float32),
    ],
)
def _sc_bin(v_hbm, a_hbm, dv_hbm, zer_hbm, out_hbm, v_v, a_v, dv_v, hist):
    _sc_bin_body(v_hbm, a_hbm, dv_hbm, zer_hbm, out_hbm, v_v, a_v, dv_v, hist)


def kernel(inclination, sky_rot, line_broadening, velocity_shift, x0, y0, distance_pc):
    f32 = jnp.float32
    cos_i = jnp.cos(inclination)
    sin_i = jnp.sin(inclination)
    pa = sky_rot + math.pi / 2.0
    cos_pa = jnp.cos(pa)
    sin_pa = jnp.sin(pa)
    arcsec_per_pc = 206265.0 / distance_pc

    sigma = jnp.abs(line_broadening) + 1e-12
    p_mid = (jnp.arange(K_VEL, dtype=f32) + 0.5) / K_VEL
    unit = math.sqrt(2.0) * jax.scipy.special.erfinv(2.0 * p_mid - 1.0)
    dv_off = sigma * unit

    params = jnp.concatenate([
        jnp.stack([cos_i, sin_i, cos_pa, sin_pa, arcsec_per_pc,
                   x0, y0, velocity_shift]).astype(f32),
        dv_off.astype(f32),
    ])

    v_los, amp = _fields(params)
    dv16 = jnp.broadcast_to(dv_off.astype(f32).reshape(K_VEL, 1), (K_VEL, 16))
    zer = jnp.zeros((_HSIZE,), f32)
    flat = _sc_bin(v_los, amp, dv16, zer)
    return (flat.reshape(_NW, NV_LO, _LROWS_W, N_PIX_LO)
            .transpose(1, 0, 2, 3)
            .reshape(NV_LO, N_PIX_LO, N_PIX_LO))


# speedup vs baseline: 267.0606x; 1.7381x over previous
import functools
import math

import jax
import jax.numpy as jnp
from jax import lax
from jax.experimental import pallas as pl
from jax.experimental.pallas import tpu as pltpu
from jax.experimental.pallas import tpu_sc as plsc

N_PIX_LO = 128
OVERSAMP_XY = 4
N_PIX_HI = N_PIX_LO * OVERSAMP_XY
NV_LO = 64
OVERSAMP_V = 4
NV_HI = NV_LO * OVERSAMP_V
K_VEL = 8
PIXSCALE_LO = 0.05
PIXSCALE_HI = PIXSCALE_LO / OVERSAMP_XY
DV_LO = 10.0
DV_HI = DV_LO / OVERSAMP_V
VEL0_LO = -0.5 * (NV_LO - 1) * DV_LO
VEL0_HI = VEL0_LO - 0.5 * (DV_LO - DV_HI)
FOV_HALF_HI = 0.5 * (N_PIX_HI - 1) * PIXSCALE_HI
THETA_E = 1.0
R_D = 500.0
V_MAX = 200.0
R_T = 200.0

_NC = 2
_NS = 16
_NW = _NC * _NS
_ROWS_W = N_PIX_HI // _NW
_LROWS_W = _ROWS_W // OVERSAMP_XY
_GROUPS = _ROWS_W * (N_PIX_HI // 16)
_HSIZE = NV_LO * _LROWS_W * N_PIX_LO


def _atan_pos(z):
    t_hi = 2.414213562373095
    t_lo = 0.4142135623730950
    hi = z > t_hi
    mid = z > t_lo
    x = jnp.where(hi, -1.0 / z, jnp.where(mid, (z - 1.0) / (z + 1.0), z))
    w = jnp.where(hi, math.pi / 2.0, jnp.where(mid, math.pi / 4.0, 0.0))
    s = x * x
    p = (((8.05374449538e-2 * s - 1.38776856032e-1) * s + 1.99777106478e-1) * s
         - 3.33329491539e-1) * s * x + x
    return w + p


_FROWS = 64


def _fields_body(params_ref, v_ref, a_ref):
    i = pl.program_id(0)
    f32 = jnp.float32

    cos_i = params_ref[0]
    sin_i = params_ref[1]
    cos_pa = params_ref[2]
    sin_pa = params_ref[3]
    arcsec_per_pc = params_ref[4]
    x0 = params_ref[5]
    y0 = params_ref[6]
    vshift = params_ref[7]

    col = lax.broadcasted_iota(jnp.int32, (_FROWS, N_PIX_HI), 1).astype(f32)
    row = (lax.broadcasted_iota(jnp.int32, (_FROWS, N_PIX_HI), 0)
           + i * _FROWS).astype(f32)
    thx = -FOV_HALF_HI + PIXSCALE_HI * col
    thy = -FOV_HALF_HI + PIXSCALE_HI * row

    r = jnp.sqrt(thx * thx + thy * thy) + 1e-12
    bx = thx - THETA_E * thx / r
    by = thy - THETA_E * thy / r
    X = (bx - x0) / arcsec_per_pc
    Y = (by - y0) / arcsec_per_pc
    x_gal = cos_pa * X + sin_pa * Y
    y_gal = (-sin_pa * X + cos_pa * Y) / (cos_i + 1e-12)
    R = jnp.hypot(x_gal, y_gal)
    I_map = jnp.exp(-R / R_D)
    v_circ = V_MAX * (2.0 / math.pi) * _atan_pos(R * (1.0 / R_T))
    cos_theta = x_gal / (R + 1e-12)
    v_los = v_circ * sin_i * cos_theta + vshift
    v_ref[...] = (v_los - VEL0_HI) * (1.0 / DV_HI)
    a_ref[...] = I_map * (1.0 / (K_VEL * OVERSAMP_V * OVERSAMP_XY * OVERSAMP_XY))


def _fields(params):
    f32 = jnp.float32
    return pl.pallas_call(
        _fields_body,
        grid=(N_PIX_HI // _FROWS,),
        in_specs=[pl.BlockSpec(memory_space=pltpu.SMEM)],
        out_specs=[
            pl.BlockSpec((_FROWS, N_PIX_HI), lambda i: (i, 0)),
            pl.BlockSpec((_FROWS, N_PIX_HI), lambda i: (i, 0)),
        ],
        out_shape=[
            jax.ShapeDtypeStruct((N_PIX_HI, N_PIX_HI), f32),
            jax.ShapeDtypeStruct((N_PIX_HI, N_PIX_HI), f32),
        ],
    )(params)


def _sc_bin_body(v_hbm, a_hbm, dv_hbm, zer_hbm, out_hbm, v_v, a_v, dv_v, hist):
    f32 = jnp.float32
    i32 = jnp.int32
    wid = lax.axis_index("s") * _NC + lax.axis_index("c")
    row0 = wid * _ROWS_W

    pltpu.sync_copy(v_hbm.at[pl.ds(row0, _ROWS_W), :], v_v)
    pltpu.sync_copy(a_hbm.at[pl.ds(row0, _ROWS_W), :], a_v)
    pltpu.sync_copy(dv_hbm, dv_v)
    pltpu.sync_copy(zer_hbm, hist)

    lane = lax.broadcasted_iota(i32, (16,), 0)
    aks = [dv_v[k, :] for k in range(K_VEL)]

    @plsc.parallel_loop(0, _GROUPS, 1, unroll=2)
    def group(g):
        hi_row = g >> 5
        x0 = pl.multiple_of((g & 31) << 4, 16)
        base = (((hi_row >> 2) << 7) + ((x0 + lane) >> 2)).astype(i32)
        v = v_v[hi_row, pl.ds(x0, 16)]
        a = a_v[hi_row, pl.ds(x0, 16)]
        for k in range(K_VEL):
            c = v + aks[k]
            iv0 = jnp.clip(c.astype(i32), 0, NV_HI - 1)
            fv = jnp.clip(c - iv0.astype(f32), 0.0, 1.0)
            idx0 = ((iv0 >> 2) << 9) + base
            idx1 = ((jnp.minimum(iv0 + 1, NV_HI - 1) >> 2) << 9) + base
            w1 = a * fv
            plsc.addupdate_scatter(hist, [idx0], a - w1)
            plsc.addupdate_scatter(hist, [idx1], w1)

    pltpu.sync_copy(hist, out_hbm.at[pl.ds(wid * _HSIZE, _HSIZE)])


@functools.partial(
    pl.kernel,
    mesh=plsc.VectorSubcoreMesh(core_axis_name="c", subcore_axis_name="s"),
    compiler_params=pltpu.CompilerParams(needs_layout_passes=False),
    out_type=jax.ShapeDtypeStruct((_NW * _HSIZE,), jnp.float32),
    scratch_types=[
        pltpu.VMEM((_ROWS_W, N_PIX_HI), jnp.float32),
        pltpu.VMEM((_ROWS_W, N_PIX_HI), jnp.float32),
        pltpu.VMEM((K_VEL, 16), jnp.float32),
        pltpu.VMEM((_HSIZE,), jnp.float32),
    ],
)
def _sc_bin(v_hbm, a_hbm, dv_hbm, zer_hbm, out_hbm, v_v, a_v, dv_v, hist):
    _sc_bin_body(v_hbm, a_hbm, dv_hbm, zer_hbm, out_hbm, v_v, a_v, dv_v, hist)


def kernel(inclination, sky_rot, line_broadening, velocity_shift, x0, y0, distance_pc):
    f32 = jnp.float32
    cos_i = jnp.cos(inclination)
    sin_i = jnp.sin(inclination)
    pa = sky_rot + math.pi / 2.0
    cos_pa = jnp.cos(pa)
    sin_pa = jnp.sin(pa)
    arcsec_per_pc = 206265.0 / distance_pc

    sigma = jnp.abs(line_broadening) + 1e-12
    p_mid = (jnp.arange(K_VEL, dtype=f32) + 0.5) / K_VEL
    unit = math.sqrt(2.0) * jax.scipy.special.erfinv(2.0 * p_mid - 1.0)
    dv_off = sigma * unit

    params = jnp.concatenate([
        jnp.stack([cos_i, sin_i, cos_pa, sin_pa, arcsec_per_pc,
                   x0, y0, velocity_shift]).astype(f32),
        dv_off.astype(f32),
    ])

    v_los, amp = _fields(params)
    dv16 = jnp.broadcast_to((dv_off * (1.0 / DV_HI)).astype(f32).reshape(K_VEL, 1),
                            (K_VEL, 16))
    zer = jnp.zeros((_HSIZE,), f32)
    flat = _sc_bin(v_los, amp, dv16, zer)
    return (flat.reshape(_NW, NV_LO, _LROWS_W, N_PIX_LO)
            .transpose(1, 0, 2, 3)
            .reshape(NV_LO, N_PIX_LO, N_PIX_LO))
